# baseline (speedup 1.0000x reference)
import functools

import jax
import jax.numpy as jnp
from jax import lax
from jax.experimental import pallas as pl
from jax.experimental.pallas import tpu as pltpu
from jax.experimental.pallas import tpu_sc as plsc

_N_EDGES = 6400000
_NW = 32
_BLK = 128
_E_CHUNK = 2048
_IDX_CHUNK = 2 * _E_CHUNK
_BLK_PER_CHUNK = _E_CHUNK // _BLK
_N_CHUNKS = _N_EDGES // _E_CHUNK
_N_PAIRS = _N_CHUNKS // 2
_TAIL_CHUNK = 2 * _N_PAIRS
_BATCHES_PER_BLK = _BLK // 16


def _compute_chunk(rows_ref, acc):
    @plsc.parallel_loop(0, _E_CHUNK // 16, unroll=2, carry=acc)
    def batch_body(b, acc2):
        lane = lax.iota(jnp.int32, 16)
        r0 = ((b >> 3) << 8) + ((b & 7) << 4)
        ri = r0 + lane
        comp = []
        for half in (0, _BLK):
            for col in range(6):
                cvec = jnp.full((16,), col, jnp.int32)
                comp.append(plsc.load_gather(rows_ref, [ri + half, cvec]))
        pxi, pyi, pzi, txi, tyi, tzi, pxj, pyj, pzj, txj, tyj, tzj = comp
        dpx = pxi - pxj
        dpy = pyi - pyj
        dpz = pzi - pzj
        dtx = txi - txj
        dty = tyi - tyj
        dtz = tzi - tzj
        dp2 = dpx * dpx + dpy * dpy + dpz * dpz
        dt2 = dtx * dtx + dty * dty + dtz * dtz
        z = dp2 * dt2
        zi = lax.bitcast_convert_type(z, jnp.int32)
        yi = jnp.int32(0x5F3759DF) - lax.shift_right_arithmetic(zi, 1)
        y = lax.bitcast_convert_type(yi, jnp.float32)
        half_z = 0.5 * z
        for _ in range(3):
            y = y * (1.5 - half_z * y * y)
        sq = z * y
        return acc2 + (dp2 + dt2 - 2.0 * sq)

    return batch_body


def _sc_body(
    table_hbm, idx_hbm, out_hbm,
    idx0, idx1, rows0, rows1, acc_v,
    sem_i0, sem_i1, sem_g0, sem_g1,
):
    cid = lax.axis_index("c")
    sid = lax.axis_index("s")
    wid = sid * 2 + cid
    p_lo = (_N_PAIRS * wid) // _NW
    p_hi = (_N_PAIRS * (wid + 1)) // _NW

    def idx_src(c):
        c = jnp.minimum(c, _N_CHUNKS - 1)
        return idx_hbm.at[pl.ds(c * _IDX_CHUNK, _IDX_CHUNK)]

    c0 = 2 * p_lo
    pltpu.sync_copy(idx_src(c0), idx0)
    pltpu.async_copy(table_hbm.at[idx0], rows0, sem_g0)
    pltpu.async_copy(idx_src(c0 + 1), idx1, sem_i1)

    def pair_body(p, acc):
        c = 2 * p
        pltpu.make_async_copy(table_hbm.at[idx0], rows0, sem_g0).wait()
        pltpu.make_async_copy(idx_src(c + 1), idx1, sem_i1).wait()
        pltpu.async_copy(table_hbm.at[idx1], rows1, sem_g1)
        pltpu.async_copy(idx_src(c + 2), idx0, sem_i0)
        acc = _compute_chunk(rows0, acc)
        pltpu.make_async_copy(idx_src(c + 2), idx0, sem_i0).wait()
        pltpu.async_copy(table_hbm.at[idx0], rows0, sem_g0)
        pltpu.make_async_copy(table_hbm.at[idx1], rows1, sem_g1).wait()
        pltpu.async_copy(idx_src(c + 3), idx1, sem_i1)
        acc = _compute_chunk(rows1, acc)
        return acc

    acc = lax.fori_loop(p_lo, p_hi, pair_body, jnp.zeros((16,), jnp.float32))

    pltpu.make_async_copy(table_hbm.at[idx0], rows0, sem_g0).wait()
    pltpu.make_async_copy(idx_src(2 * p_hi + 1), idx1, sem_i1).wait()
    tail = _compute_chunk(rows0, jnp.zeros((16,), jnp.float32))
    is_tail = (2 * p_hi == _TAIL_CHUNK).astype(jnp.float32)
    acc = acc + jnp.full((16,), 1.0, jnp.float32) * is_tail * tail

    acc_v[...] = acc * (1.0 / _N_EDGES)
    pltpu.sync_copy(acc_v, out_hbm.at[wid])


def kernel(pred_coords, true_coords, bond_pairs):
    n = pred_coords.shape[0]
    table = jnp.concatenate(
        [pred_coords, true_coords, jnp.zeros((n, 2), jnp.float32)], axis=1
    )
    idx = (
        bond_pairs.astype(jnp.int32)
        .reshape(-1, _BLK, 2)
        .transpose(0, 2, 1)
        .reshape(-1)
    )
    mesh = plsc.VectorSubcoreMesh(core_axis_name="c", subcore_axis_name="s")
    run = functools.partial(
        pl.kernel,
        mesh=mesh,
        out_type=jax.ShapeDtypeStruct((_NW, 16), jnp.float32),
        scratch_types=[
            pltpu.VMEM((_IDX_CHUNK,), jnp.int32),
            pltpu.VMEM((_IDX_CHUNK,), jnp.int32),
            pltpu.VMEM((_IDX_CHUNK, 8), jnp.float32),
            pltpu.VMEM((_IDX_CHUNK, 8), jnp.float32),
            pltpu.VMEM((16,), jnp.float32),
            pltpu.SemaphoreType.DMA,
            pltpu.SemaphoreType.DMA,
            pltpu.SemaphoreType.DMA,
            pltpu.SemaphoreType.DMA,
        ],
        compiler_params=pltpu.CompilerParams(
            needs_layout_passes=False, use_tc_tiling_on_sc=False
        ),
    )(_sc_body)
    partials = run(table, idx)
    return jnp.sum(partials)


# speedup vs baseline: 1.0398x; 1.0398x over previous
import functools

import jax
import jax.numpy as jnp
from jax import lax
from jax.experimental import pallas as pl
from jax.experimental.pallas import tpu as pltpu
from jax.experimental.pallas import tpu_sc as plsc

_N_EDGES = 6400000
_NW = 32
_BLK = 128
_E_CHUNK = 2048
_IDX_CHUNK = 2 * _E_CHUNK
_BLK_PER_CHUNK = _E_CHUNK // _BLK
_N_CHUNKS = _N_EDGES // _E_CHUNK
_N_PAIRS = _N_CHUNKS // 2
_TAIL_CHUNK = 2 * _N_PAIRS
_BATCHES_PER_BLK = _BLK // 16


def _compute_chunk(rows_ref, acc):
    def blk_body(blk, acc2):
        lane = lax.iota(jnp.int32, 16)
        row_i0 = 2 * _BLK * blk + lane
        for s in range(_BATCHES_PER_BLK):
            ri = row_i0 + 16 * s
            comp = []
            for half in (0, _BLK):
                for col in range(6):
                    cvec = jnp.full((16,), col, jnp.int32)
                    comp.append(plsc.load_gather(rows_ref, [ri + half, cvec]))
            pxi, pyi, pzi, txi, tyi, tzi, pxj, pyj, pzj, txj, tyj, tzj = comp
            dpx = pxi - pxj
            dpy = pyi - pyj
            dpz = pzi - pzj
            dtx = txi - txj
            dty = tyi - tyj
            dtz = tzi - tzj
            dp2 = dpx * dpx + dpy * dpy + dpz * dpz
            dt2 = dtx * dtx + dty * dty + dtz * dtz
            z = dp2 * dt2
            zi = lax.bitcast_convert_type(z, jnp.int32)
            yi = jnp.int32(0x5F3759DF) - lax.shift_right_arithmetic(zi, 1)
            y = lax.bitcast_convert_type(yi, jnp.float32)
            half_z = 0.5 * z
            for _ in range(3):
                y = y * (1.5 - half_z * y * y)
            sq = z * y
            acc2 = acc2 + (dp2 + dt2 - 2.0 * sq)
        return acc2

    return lax.fori_loop(0, _BLK_PER_CHUNK, blk_body, acc)


def _sc_body(
    table_hbm, idx_hbm, out_hbm,
    idx0, idx1, rows0, rows1, acc_v,
    sem_i0, sem_i1, sem_g0, sem_g1,
):
    cid = lax.axis_index("c")
    sid = lax.axis_index("s")
    wid = sid * 2 + cid
    p_lo = (_N_PAIRS * wid) // _NW
    p_hi = (_N_PAIRS * (wid + 1)) // _NW

    def idx_src(c):
        c = jnp.minimum(c, _N_CHUNKS - 1)
        return idx_hbm.at[pl.ds(c * _IDX_CHUNK, _IDX_CHUNK)]

    c0 = 2 * p_lo
    pltpu.sync_copy(idx_src(c0), idx0)
    pltpu.async_copy(table_hbm.at[idx0], rows0, sem_g0)
    pltpu.async_copy(idx_src(c0 + 1), idx1, sem_i1)

    def pair_body(p, acc):
        c = 2 * p
        pltpu.make_async_copy(table_hbm.at[idx0], rows0, sem_g0).wait()
        pltpu.make_async_copy(idx_src(c + 1), idx1, sem_i1).wait()
        pltpu.async_copy(table_hbm.at[idx1], rows1, sem_g1)
        pltpu.async_copy(idx_src(c + 2), idx0, sem_i0)
        acc = _compute_chunk(rows0, acc)
        pltpu.make_async_copy(idx_src(c + 2), idx0, sem_i0).wait()
        pltpu.async_copy(table_hbm.at[idx0], rows0, sem_g0)
        pltpu.make_async_copy(table_hbm.at[idx1], rows1, sem_g1).wait()
        pltpu.async_copy(idx_src(c + 3), idx1, sem_i1)
        acc = _compute_chunk(rows1, acc)
        return acc

    acc = lax.fori_loop(p_lo, p_hi, pair_body, jnp.zeros((16,), jnp.float32))

    pltpu.make_async_copy(table_hbm.at[idx0], rows0, sem_g0).wait()
    pltpu.make_async_copy(idx_src(2 * p_hi + 1), idx1, sem_i1).wait()
    tail = _compute_chunk(rows0, jnp.zeros((16,), jnp.float32))
    is_tail = (2 * p_hi == _TAIL_CHUNK).astype(jnp.float32)
    acc = acc + is_tail * tail

    acc_v[...] = acc * (1.0 / _N_EDGES)
    pltpu.sync_copy(acc_v, out_hbm.at[wid])


def kernel(pred_coords, true_coords, bond_pairs):
    n = pred_coords.shape[0]
    table = jnp.concatenate(
        [pred_coords, true_coords, jnp.zeros((n, 2), jnp.float32)], axis=1
    )
    idx = (
        bond_pairs.astype(jnp.int32)
        .reshape(-1, _BLK, 2)
        .transpose(0, 2, 1)
        .reshape(-1)
    )
    mesh = plsc.VectorSubcoreMesh(core_axis_name="c", subcore_axis_name="s")
    run = functools.partial(
        pl.kernel,
        mesh=mesh,
        out_type=jax.ShapeDtypeStruct((_NW, 16), jnp.float32),
        scratch_types=[
            pltpu.VMEM((_IDX_CHUNK,), jnp.int32),
            pltpu.VMEM((_IDX_CHUNK,), jnp.int32),
            pltpu.VMEM((_IDX_CHUNK, 8), jnp.float32),
            pltpu.VMEM((_IDX_CHUNK, 8), jnp.float32),
            pltpu.VMEM((16,), jnp.float32),
            pltpu.SemaphoreType.DMA,
            pltpu.SemaphoreType.DMA,
            pltpu.SemaphoreType.DMA,
            pltpu.SemaphoreType.DMA,
        ],
        compiler_params=pltpu.CompilerParams(
            needs_layout_passes=False,
            use_tc_tiling_on_sc=False,
            disable_bounds_checks=True,
        ),
    )(_sc_body)
    partials = run(table, idx)
    return jnp.sum(partials)


# speedup vs baseline: 1.1621x; 1.1176x over previous
import functools

import jax
import jax.numpy as jnp
from jax import lax
from jax.experimental import pallas as pl
from jax.experimental.pallas import tpu as pltpu
from jax.experimental.pallas import tpu_sc as plsc

_N_EDGES = 6400000
_N_NODES = 100000
_NW = 32
_BLK = 128
_E_CHUNK = 2048
_IDX_CHUNK = 2 * _E_CHUNK
_BLK_PER_CHUNK = _E_CHUNK // _BLK
_N_CHUNKS = _N_EDGES // _E_CHUNK
_N_PAIRS = _N_CHUNKS // 2
_TAIL_CHUNK = 2 * _N_PAIRS
_BATCHES_PER_BLK = _BLK // 16


def _compute_chunk(rows_ref, acc):
    def blk_body(blk, acc2):
        lane = lax.iota(jnp.int32, 16)
        row_i0 = 2 * _BLK * blk + lane
        for s in range(_BATCHES_PER_BLK):
            ri = row_i0 + 16 * s
            comp = []
            for half in (0, _BLK):
                for col in range(6):
                    cvec = jnp.full((16,), col, jnp.int32)
                    comp.append(plsc.load_gather(rows_ref, [ri + half, cvec]))
            pxi, pyi, pzi, txi, tyi, tzi, pxj, pyj, pzj, txj, tyj, tzj = comp
            dpx = pxi - pxj
            dpy = pyi - pyj
            dpz = pzi - pzj
            dtx = txi - txj
            dty = tyi - tyj
            dtz = tzi - tzj
            dp2 = dpx * dpx + dpy * dpy + dpz * dpz
            dt2 = dtx * dtx + dty * dty + dtz * dtz
            z = dp2 * dt2
            zi = lax.bitcast_convert_type(z, jnp.int32)
            yi = jnp.int32(0x5F3759DF) - lax.shift_right_arithmetic(zi, 1)
            y = lax.bitcast_convert_type(yi, jnp.float32)
            half_z = 0.5 * z
            for _ in range(3):
                y = y * (1.5 - half_z * y * y)
            sq = z * y
            acc2 = acc2 + (dp2 + dt2 - 2.0 * sq)
        return acc2

    return lax.fori_loop(0, _BLK_PER_CHUNK, blk_body, acc)


def _sc_body(
    table_hbm, idx_hbm, out_hbm,
    tbl_s, idx0, idx1, rows0, rows1, acc_v,
    sem_i0, sem_i1, sem_g0, sem_g1,
):
    cid = lax.axis_index("c")
    sid = lax.axis_index("s")
    wid = sid * 2 + cid
    p_lo = (_N_PAIRS * wid) // _NW
    p_hi = (_N_PAIRS * (wid + 1)) // _NW

    rows_per_tile = _N_NODES // 16
    s0 = sid * rows_per_tile
    pltpu.sync_copy(
        table_hbm.at[pl.ds(s0, rows_per_tile)], tbl_s.at[pl.ds(s0, rows_per_tile)]
    )
    plsc.subcore_barrier()

    def idx_src(c):
        c = jnp.minimum(c, _N_CHUNKS - 1)
        return idx_hbm.at[pl.ds(c * _IDX_CHUNK, _IDX_CHUNK)]

    table_src = tbl_s

    c0 = 2 * p_lo
    pltpu.sync_copy(idx_src(c0), idx0)
    pltpu.async_copy(table_src.at[idx0], rows0, sem_g0)
    pltpu.async_copy(idx_src(c0 + 1), idx1, sem_i1)

    def pair_body(p, acc):
        c = 2 * p
        pltpu.make_async_copy(table_src.at[idx0], rows0, sem_g0).wait()
        pltpu.make_async_copy(idx_src(c + 1), idx1, sem_i1).wait()
        pltpu.async_copy(table_src.at[idx1], rows1, sem_g1)
        pltpu.async_copy(idx_src(c + 2), idx0, sem_i0)
        acc = _compute_chunk(rows0, acc)
        pltpu.make_async_copy(idx_src(c + 2), idx0, sem_i0).wait()
        pltpu.async_copy(table_src.at[idx0], rows0, sem_g0)
        pltpu.make_async_copy(table_src.at[idx1], rows1, sem_g1).wait()
        pltpu.async_copy(idx_src(c + 3), idx1, sem_i1)
        acc = _compute_chunk(rows1, acc)
        return acc

    acc = lax.fori_loop(p_lo, p_hi, pair_body, jnp.zeros((16,), jnp.float32))

    pltpu.make_async_copy(table_src.at[idx0], rows0, sem_g0).wait()
    pltpu.make_async_copy(idx_src(2 * p_hi + 1), idx1, sem_i1).wait()
    tail = _compute_chunk(rows0, jnp.zeros((16,), jnp.float32))
    is_tail = (2 * p_hi == _TAIL_CHUNK).astype(jnp.float32)
    acc = acc + is_tail * tail

    acc_v[...] = acc * (1.0 / _N_EDGES)
    pltpu.sync_copy(acc_v, out_hbm.at[wid])


def kernel(pred_coords, true_coords, bond_pairs):
    n = pred_coords.shape[0]
    table = jnp.concatenate(
        [pred_coords, true_coords, jnp.zeros((n, 2), jnp.float32)], axis=1
    )
    idx = (
        bond_pairs.astype(jnp.int32)
        .reshape(-1, _BLK, 2)
        .transpose(0, 2, 1)
        .reshape(-1)
    )
    mesh = plsc.VectorSubcoreMesh(core_axis_name="c", subcore_axis_name="s")
    run = functools.partial(
        pl.kernel,
        mesh=mesh,
        out_type=jax.ShapeDtypeStruct((_NW, 16), jnp.float32),
        scratch_types=[
            pltpu.VMEM_SHARED((_N_NODES, 8), jnp.float32),
            pltpu.VMEM((_IDX_CHUNK,), jnp.int32),
            pltpu.VMEM((_IDX_CHUNK,), jnp.int32),
            pltpu.VMEM((_IDX_CHUNK, 8), jnp.float32),
            pltpu.VMEM((_IDX_CHUNK, 8), jnp.float32),
            pltpu.VMEM((16,), jnp.float32),
            pltpu.SemaphoreType.DMA,
            pltpu.SemaphoreType.DMA,
            pltpu.SemaphoreType.DMA,
            pltpu.SemaphoreType.DMA,
        ],
        compiler_params=pltpu.CompilerParams(
            needs_layout_passes=False,
            use_tc_tiling_on_sc=False,
            disable_bounds_checks=True,
        ),
    )(_sc_body)
    partials = run(table, idx)
    return jnp.sum(partials)
